# baseline (speedup 1.0000x reference)
import functools

import jax
import jax.numpy as jnp
from jax import lax
from jax.experimental import pallas as pl
from jax.experimental.pallas import tpu as pltpu
from jax.experimental.pallas import tpu_sc as plsc

EMBED = 16
BATCH = 16384
LANES = 16
NW = 32
BPW = BATCH // NW
NCHUNK = 4
CHUNK = BPW // NCHUNK
NGROUP = BPW // LANES


def _sc_call(idx, table, w, bvec):
    mesh = plsc.VectorSubcoreMesh(core_axis_name="c", subcore_axis_name="s")

    @functools.partial(
        pl.kernel,
        mesh=mesh,
        compiler_params=pltpu.CompilerParams(
            needs_layout_passes=False, use_tc_tiling_on_sc=False
        ),
        out_type=jax.ShapeDtypeStruct((NW, BPW), jnp.float32),
        scratch_types=[
            pltpu.VMEM((NCHUNK, CHUNK), jnp.int32),
            pltpu.VMEM((BPW, EMBED), jnp.float32),
            pltpu.VMEM((LANES,), jnp.float32),
            pltpu.VMEM((LANES,), jnp.float32),
            pltpu.VMEM((BPW,), jnp.float32),
            pltpu.SemaphoreType.DMA,
        ],
    )
    def sc_kernel(idx_hbm, table_hbm, w_hbm, b_hbm, out_hbm,
                  idx_v, rows_v, w_v, b_v, out_v, sem):
        wid = lax.axis_index("s") * 2 + lax.axis_index("c")
        pltpu.sync_copy(idx_hbm.at[wid], idx_v)
        pltpu.sync_copy(w_hbm, w_v)
        pltpu.sync_copy(b_hbm, b_v)

        copies = [
            pltpu.async_copy(
                table_hbm.at[idx_v.at[c]],
                rows_v.at[pl.ds(c * CHUNK, CHUNK)],
                sem,
            )
            for c in range(NCHUNK)
        ]
        for cp in copies:
            cp.wait()

        w = w_v[...]
        b16 = b_v[...]
        last_lane = lax.iota(jnp.int32, LANES) == (LANES - 1)

        def body(g, carry):
            for k in range(LANES):
                row = rows_v[g * LANES + k, :]
                prod = row * w + b16
                csum = lax.cumsum(prod, axis=0)
                pos = jnp.full((LANES,), g * LANES + k, jnp.int32)
                plsc.store_scatter(out_v, [pos], csum, mask=last_lane)
            return carry

        lax.fori_loop(0, NGROUP, body, 0)
        pltpu.sync_copy(out_v, out_hbm.at[wid])

    return sc_kernel(idx, table, w, bvec)


def kernel(x, table, W, b):
    idx = x.reshape(NW, NCHUNK, CHUNK).astype(jnp.int32)
    w = W.astype(jnp.float32).reshape(EMBED)
    bvec = jnp.broadcast_to(
        (b.astype(jnp.float32) / LANES).reshape(1), (LANES,)
    )
    out = _sc_call(idx, table.astype(jnp.float32), w, bvec)
    return out.reshape(BATCH, 1)


# speedup vs baseline: 2.4064x; 2.4064x over previous
import functools

import jax
import jax.numpy as jnp
from jax import lax
from jax.experimental import pallas as pl
from jax.experimental.pallas import tpu as pltpu
from jax.experimental.pallas import tpu_sc as plsc

VOCAB = 1000000
EMBED = 16
BATCH = 16384
LANES = 16
NW = 32
BPW = BATCH // NW
CHUNK = 32
NCHUNK = BPW // CHUNK
GPC = CHUNK // LANES


def _sc_call(idx, table3d, wb, bvec):
    mesh = plsc.VectorSubcoreMesh(core_axis_name="c", subcore_axis_name="s")

    @functools.partial(
        pl.kernel,
        mesh=mesh,
        compiler_params=pltpu.CompilerParams(needs_layout_passes=False),
        out_type=jax.ShapeDtypeStruct((NW, BPW), jnp.float32),
        scratch_types=[
            pltpu.VMEM((BPW,), jnp.int32),
            pltpu.VMEM((BPW,), jnp.int32),
            pltpu.VMEM((BPW,), jnp.int32),
            pltpu.VMEM((CHUNK, 8, EMBED), jnp.float32),
            pltpu.VMEM((CHUNK, 8, EMBED), jnp.float32),
            pltpu.VMEM((EMBED, LANES), jnp.float32),
            pltpu.VMEM((LANES,), jnp.float32),
            pltpu.VMEM((BPW,), jnp.float32),
            pltpu.SemaphoreType.DMA,
            pltpu.SemaphoreType.DMA,
        ],
    )
    def sc_kernel(idx_hbm, table_hbm, wb_hbm, b_hbm, out_hbm,
                  x_v, tid_v, sid_v, buf_a, buf_b, wb_v, b_v, out_v,
                  sem_a, sem_b):
        wid = lax.axis_index("s") * 2 + lax.axis_index("c")
        pltpu.sync_copy(idx_hbm.at[wid], x_v)
        pltpu.sync_copy(wb_hbm, wb_v)
        pltpu.sync_copy(b_hbm, b_v)

        for u in range(BPW // LANES):
            xv = x_v[pl.ds(u * LANES, LANES)]
            tid_v[pl.ds(u * LANES, LANES)] = lax.shift_right_logical(
                xv, jnp.full((LANES,), 3, jnp.int32)
            )
            sid_v[pl.ds(u * LANES, LANES)] = lax.bitwise_and(
                xv, jnp.full((LANES,), 7, jnp.int32)
            )

        bufs = (buf_a, buf_b)
        sems = (sem_a, sem_b)

        def fire(c):
            cps = []
            for u in range(CHUNK // LANES):
                tv = tid_v[pl.ds(c * CHUNK + u * LANES, LANES)]
                for k in range(LANES):
                    cps.append(
                        pltpu.async_copy(
                            table_hbm.at[tv[k]],
                            bufs[c % 2].at[u * LANES + k],
                            sems[c % 2],
                        )
                    )
            return cps

        wrows = [wb_v[j, :] for j in range(EMBED)]
        bias = b_v[...]
        base_iota = lax.iota(jnp.int32, LANES)

        pending = fire(0)
        for c in range(NCHUNK):
            nxt = fire(c + 1) if c + 1 < NCHUNK else None
            for cp in pending:
                cp.wait()
            buf = bufs[c % 2]
            for g in range(GPC):
                off = c * CHUNK + g * LANES
                i_ids = base_iota + (g * LANES)
                s_ids = sid_v[pl.ds(off, LANES)]
                acc = bias
                for j in range(EMBED):
                    col = plsc.load_gather(
                        buf, [i_ids, s_ids, jnp.full((LANES,), j, jnp.int32)]
                    )
                    acc = acc + col * wrows[j]
                out_v[pl.ds(off, LANES)] = acc
            pending = nxt

        pltpu.sync_copy(out_v, out_hbm.at[wid])

    return sc_kernel(idx, table3d, wb, bvec)


def kernel(x, table, W, b):
    idx = x.reshape(NW, BPW).astype(jnp.int32)
    table3d = table.astype(jnp.float32).reshape(VOCAB // 8, 8, EMBED)
    wb = jnp.broadcast_to(
        W.astype(jnp.float32).reshape(EMBED, 1), (EMBED, LANES)
    )
    bvec = jnp.broadcast_to(b.astype(jnp.float32).reshape(1), (LANES,))
    out = _sc_call(idx, table3d, wb, bvec)
    return out.reshape(BATCH, 1)
